# baseline (speedup 1.0000x reference)
import functools

import jax
import jax.numpy as jnp
from jax import lax
from jax.experimental import pallas as pl
from jax.experimental.pallas import tpu as pltpu
from jax.experimental.pallas import tpu_sc as plsc

VOCAB = 100000
EMB = 64
BATCH = 1024

NV = 2048


@functools.lru_cache(maxsize=None)
def _make_sc_gather():
  info = plsc.get_sparse_core_info()
  nc, ns = info.num_cores, info.num_subcores
  nw = nc * ns
  b_per_w = BATCH // nw
  mesh = plsc.VectorSubcoreMesh(core_axis_name="c", subcore_axis_name="s")

  @functools.partial(
      pl.kernel,
      mesh=mesh,
      out_type=jax.ShapeDtypeStruct((BATCH, EMB), jnp.float32),
      scratch_types=[
          pltpu.VMEM((b_per_w,), jnp.int32),
          pltpu.VMEM((b_per_w, EMB), jnp.float32),
          pltpu.SemaphoreType.DMA,
      ],
      compiler_params=pltpu.CompilerParams(use_tc_tiling_on_sc=False),
  )
  def gather(table_hbm, idx_hbm, out_hbm, idx_v, rows_v, sem):
    wid = lax.axis_index("s") * nc + lax.axis_index("c")
    base = wid * b_per_w
    pltpu.sync_copy(idx_hbm.at[pl.ds(base, b_per_w)], idx_v)
    pltpu.async_copy(table_hbm.at[idx_v], rows_v, sem).wait()
    pltpu.sync_copy(rows_v, out_hbm.at[pl.ds(base, b_per_w)])

  return gather


def _proj_kernel(x_ref, w_ref, b_ref, o_ref):
  o_ref[...] = lax.dot_general(
      x_ref[...], w_ref[...],
      dimension_numbers=(((1,), (1,)), ((), ())),
      preferred_element_type=jnp.float32,
  ) + b_ref[...]


@jax.jit
def kernel(contexts, embedding, W, b):
  x = _make_sc_gather()(embedding, contexts.astype(jnp.int32))

  grid = pl.cdiv(VOCAB, NV)
  out = pl.pallas_call(
      _proj_kernel,
      grid=(grid,),
      in_specs=[
          pl.BlockSpec((BATCH, EMB), lambda i: (0, 0)),
          pl.BlockSpec((NV, EMB), lambda i: (i, 0)),
          pl.BlockSpec((1, NV), lambda i: (0, i)),
      ],
      out_specs=pl.BlockSpec((BATCH, NV), lambda i: (0, i)),
      out_shape=jax.ShapeDtypeStruct((BATCH, VOCAB), jnp.float32),
  )(x, W, b.reshape(1, VOCAB))
  return out


# speedup vs baseline: 1.0557x; 1.0557x over previous
import functools

import jax
import jax.numpy as jnp
from jax import lax
from jax.experimental import pallas as pl
from jax.experimental.pallas import tpu as pltpu
from jax.experimental.pallas import tpu_sc as plsc

VOCAB = 100000
EMB = 64
BATCH = 1024

NV = 2048


@functools.lru_cache(maxsize=None)
def _make_sc_gather():
  info = plsc.get_sparse_core_info()
  nc, ns, nl = info.num_cores, info.num_subcores, info.num_lanes
  nw = nc * ns
  b_per_w = BATCH // nw
  mesh = plsc.VectorSubcoreMesh(core_axis_name="c", subcore_axis_name="s")

  @functools.partial(
      pl.kernel,
      mesh=mesh,
      out_type=jax.ShapeDtypeStruct((BATCH, EMB), jnp.float32),
      scratch_types=[
          pltpu.VMEM((b_per_w,), jnp.int32),
          pltpu.VMEM((b_per_w, EMB), jnp.float32),
          pltpu.SemaphoreType.DMA,
      ],
      compiler_params=pltpu.CompilerParams(needs_layout_passes=False),
  )
  def gather(table_hbm, idx_hbm, out_hbm, idx_v, rows_v, sem):
    wid = lax.axis_index("s") * nc + lax.axis_index("c")
    base = wid * b_per_w
    pltpu.sync_copy(idx_hbm.at[pl.ds(base, b_per_w)], idx_v)
    lane = lax.broadcasted_iota(jnp.int32, (nl,), 0)
    copies = []
    for j in range(b_per_w):
      vec = idx_v[pl.ds((j // nl) * nl, nl)]
      row = lax.reduce_max(
          jnp.where(lane == (j % nl), vec, 0), axes=(0,))
      copies.append(pltpu.async_copy(table_hbm.at[row], rows_v.at[j], sem))
    for c in copies:
      c.wait()
    pltpu.sync_copy(rows_v, out_hbm.at[pl.ds(base, b_per_w)])

  return gather


def _proj_kernel(x_ref, w_ref, b_ref, o_ref):
  o_ref[...] = lax.dot_general(
      x_ref[...], w_ref[...],
      dimension_numbers=(((1,), (1,)), ((), ())),
      preferred_element_type=jnp.float32,
  ) + b_ref[...][None, :]


@jax.jit
def kernel(contexts, embedding, W, b):
  x = _make_sc_gather()(embedding, contexts.astype(jnp.int32))

  grid = pl.cdiv(VOCAB, NV)
  out = pl.pallas_call(
      _proj_kernel,
      grid=(grid,),
      in_specs=[
          pl.BlockSpec((BATCH, EMB), lambda i: (0, 0)),
          pl.BlockSpec((NV, EMB), lambda i: (i, 0)),
          pl.BlockSpec((NV,), lambda i: (i,)),
      ],
      out_specs=pl.BlockSpec((BATCH, NV), lambda i: (0, i)),
      out_shape=jax.ShapeDtypeStruct((BATCH, VOCAB), jnp.float32),
  )(x, W, b)
  return out


# speedup vs baseline: 2.8279x; 2.6788x over previous
import functools

import jax
import jax.numpy as jnp
from jax import lax
from jax.experimental import pallas as pl
from jax.experimental.pallas import tpu as pltpu
from jax.experimental.pallas import tpu_sc as plsc

VOCAB = 100000
EMB = 64
BATCH = 1024

NV = 2048


@functools.lru_cache(maxsize=None)
def _make_sc_gather():
  info = plsc.get_sparse_core_info()
  nc, ns = info.num_cores, info.num_subcores
  nw = nc * ns
  b_per_w = BATCH // nw
  mesh = plsc.VectorSubcoreMesh(core_axis_name="c", subcore_axis_name="s")

  @functools.partial(
      pl.kernel,
      mesh=mesh,
      out_type=jax.ShapeDtypeStruct((BATCH, EMB), jnp.float32),
      scratch_types=[
          pltpu.VMEM((b_per_w,), jnp.int32),
          pltpu.VMEM((b_per_w, EMB), jnp.float32),
          pltpu.SemaphoreType.DMA,
      ],
      compiler_params=pltpu.CompilerParams(use_tc_tiling_on_sc=False),
  )
  def gather(table_hbm, idx_hbm, out_hbm, idx_v, rows_v, sem):
    wid = lax.axis_index("s") * nc + lax.axis_index("c")
    base = wid * b_per_w
    pltpu.sync_copy(idx_hbm.at[pl.ds(base, b_per_w)], idx_v)
    pltpu.async_copy(table_hbm.at[idx_v], rows_v, sem).wait()
    pltpu.sync_copy(rows_v, out_hbm.at[pl.ds(base, b_per_w)])

  return gather


def _proj_kernel(wt_ref, x_ref, b_ref, o_ref):
  o_ref[...] = lax.dot_general(
      wt_ref[...], x_ref[...],
      dimension_numbers=(((0,), (1,)), ((), ())),
      preferred_element_type=jnp.float32,
  ) + b_ref[...][:, None]


@jax.jit
def kernel(contexts, embedding, W, b):
  x = _make_sc_gather()(embedding, contexts.astype(jnp.int32))
  Wt = W.T

  grid = pl.cdiv(VOCAB, NV)
  out_t = pl.pallas_call(
      _proj_kernel,
      grid=(grid,),
      in_specs=[
          pl.BlockSpec((EMB, NV), lambda i: (0, i)),
          pl.BlockSpec((BATCH, EMB), lambda i: (0, 0)),
          pl.BlockSpec((NV,), lambda i: (i,)),
      ],
      out_specs=pl.BlockSpec((NV, BATCH), lambda i: (i, 0)),
      out_shape=jax.ShapeDtypeStruct((VOCAB, BATCH), jnp.float32),
  )(Wt, x, b)
  return out_t.T


# speedup vs baseline: 3.5819x; 1.2666x over previous
import functools

import jax
import jax.numpy as jnp
from jax import lax
from jax.experimental import pallas as pl
from jax.experimental.pallas import tpu as pltpu
from jax.experimental.pallas import tpu_sc as plsc

VOCAB = 100000
EMB = 64
BATCH = 1024
XPAD = 128
IDXBITS = 17

NV = 2048

WAVE = 8


@functools.lru_cache(maxsize=None)
def _make_sc_gather():
  info = plsc.get_sparse_core_info()
  nc, ns, nl = info.num_cores, info.num_subcores, info.num_lanes
  nw = nc * ns
  b_per_w = BATCH // nw
  mesh = plsc.VectorSubcoreMesh(core_axis_name="c", subcore_axis_name="s")

  @functools.partial(
      pl.kernel,
      mesh=mesh,
      out_type=jax.ShapeDtypeStruct((BATCH, XPAD), jnp.float32),
      scratch_types=[
          pltpu.VMEM((BATCH,), jnp.int32),
          pltpu.VMEM((WAVE, EMB, XPAD), jnp.float32),
          pltpu.VMEM((b_per_w, XPAD), jnp.float32),
          pltpu.SemaphoreType.DMA,
      ],
      compiler_params=pltpu.CompilerParams(needs_layout_passes=False),
  )
  def gather(table_hbm, idx_hbm, out_hbm, idx_v, fetch_v, rows_v, sem):
    wid = lax.axis_index("s") * nc + lax.axis_index("c")
    base = wid * b_per_w
    pltpu.sync_copy(idx_hbm, idx_v)
    lane = lax.broadcasted_iota(jnp.int32, (nl,), 0)
    for wave in range(b_per_w // WAVE):
      copies = []
      lsplats = []
      for jw in range(WAVE):
        j = wave * WAVE + jw
        cvec = idx_v[pl.ds(base + (j // nl) * nl, nl)]
        c = cvec[j % nl]
        col0 = pl.multiple_of((c >> 7) * XPAD, XPAD)
        copies.append(
            pltpu.async_copy(
                table_hbm.at[:, pl.ds(col0, XPAD)], fetch_v.at[jw], sem))
        lsplats.append(jnp.full((nl,), c & (XPAD - 1), jnp.int32))
      for cp in copies:
        cp.wait()
      for jw in range(WAVE):
        j = wave * WAVE + jw
        for kc in range(EMB // nl):
          vals = plsc.load_gather(
              fetch_v.at[jw],
              [kc * nl + lane, lsplats[jw]])
          rows_v[j, pl.ds(kc * nl, nl)] = vals
    pltpu.sync_copy(rows_v, out_hbm.at[pl.ds(base, b_per_w)])

  return gather


def _proj_kernel(wt_ref, x_ref, b_ref, o_ref):
  o_ref[...] = lax.dot_general(
      wt_ref[...], x_ref[:, :EMB],
      dimension_numbers=(((0,), (1,)), ((), ())),
      preferred_element_type=jnp.float32,
  ) + b_ref[...][:, None]


@jax.jit
def kernel(contexts, embedding, W, b):
  x_pad = _make_sc_gather()(embedding.T, contexts.astype(jnp.int32))
  Wt = W.T

  grid = pl.cdiv(VOCAB, NV)
  out_t = pl.pallas_call(
      _proj_kernel,
      grid=(grid,),
      in_specs=[
          pl.BlockSpec((EMB, NV), lambda i: (0, i)),
          pl.BlockSpec((BATCH, XPAD), lambda i: (0, 0)),
          pl.BlockSpec((NV,), lambda i: (i,)),
      ],
      out_specs=pl.BlockSpec((NV, BATCH), lambda i: (i, 0)),
      out_shape=jax.ShapeDtypeStruct((VOCAB, BATCH), jnp.float32),
  )(Wt, x_pad, b)
  return out_t.T


# speedup vs baseline: 3.6229x; 1.0115x over previous
import functools

import jax
import jax.numpy as jnp
from jax import lax
from jax.experimental import pallas as pl
from jax.experimental.pallas import tpu as pltpu
from jax.experimental.pallas import tpu_sc as plsc

VOCAB = 100000
EMB = 64
BATCH = 1024
XPAD = 128
IDXBITS = 17

NV = 4096

WAVE = 8


@functools.lru_cache(maxsize=None)
def _make_sc_gather():
  info = plsc.get_sparse_core_info()
  nc, ns, nl = info.num_cores, info.num_subcores, info.num_lanes
  nw = nc * ns
  b_per_w = BATCH // nw
  mesh = plsc.VectorSubcoreMesh(core_axis_name="c", subcore_axis_name="s")

  @functools.partial(
      pl.kernel,
      mesh=mesh,
      out_type=jax.ShapeDtypeStruct((BATCH, XPAD), jnp.float32),
      scratch_types=[
          pltpu.VMEM((BATCH,), jnp.int32),
          pltpu.VMEM((WAVE, EMB, XPAD), jnp.float32),
          pltpu.VMEM((b_per_w, XPAD), jnp.float32),
          pltpu.SemaphoreType.DMA,
      ],
      compiler_params=pltpu.CompilerParams(needs_layout_passes=False),
  )
  def gather(table_hbm, idx_hbm, out_hbm, idx_v, fetch_v, rows_v, sem):
    wid = lax.axis_index("s") * nc + lax.axis_index("c")
    base = wid * b_per_w
    pltpu.sync_copy(idx_hbm, idx_v)
    lane = lax.broadcasted_iota(jnp.int32, (nl,), 0)
    for wave in range(b_per_w // WAVE):
      copies = []
      lsplats = []
      for jw in range(WAVE):
        j = wave * WAVE + jw
        cvec = idx_v[pl.ds(base + (j // nl) * nl, nl)]
        c = cvec[j % nl]
        col0 = pl.multiple_of((c >> 7) * XPAD, XPAD)
        copies.append(
            pltpu.async_copy(
                table_hbm.at[:, pl.ds(col0, XPAD)], fetch_v.at[jw], sem))
        lsplats.append(jnp.full((nl,), c & (XPAD - 1), jnp.int32))
      for cp in copies:
        cp.wait()
      for jw in range(WAVE):
        j = wave * WAVE + jw
        for kc in range(EMB // nl):
          vals = plsc.load_gather(
              fetch_v.at[jw],
              [kc * nl + lane, lsplats[jw]])
          rows_v[j, pl.ds(kc * nl, nl)] = vals
    pltpu.sync_copy(rows_v, out_hbm.at[pl.ds(base, b_per_w)])

  return gather


def _proj_kernel(wt_ref, x_ref, b_ref, o_ref):
  o_ref[...] = lax.dot_general(
      wt_ref[...], x_ref[:, :EMB],
      dimension_numbers=(((0,), (1,)), ((), ())),
      preferred_element_type=jnp.float32,
  ) + b_ref[...][:, None]


@jax.jit
def kernel(contexts, embedding, W, b):
  x_pad = _make_sc_gather()(embedding.T, contexts.astype(jnp.int32))
  Wt = W.T

  grid = pl.cdiv(VOCAB, NV)
  out_t = pl.pallas_call(
      _proj_kernel,
      grid=(grid,),
      in_specs=[
          pl.BlockSpec((EMB, NV), lambda i: (0, i)),
          pl.BlockSpec((BATCH, XPAD), lambda i: (0, 0)),
          pl.BlockSpec((NV,), lambda i: (i,)),
      ],
      out_specs=pl.BlockSpec((NV, BATCH), lambda i: (i, 0)),
      out_shape=jax.ShapeDtypeStruct((VOCAB, BATCH), jnp.float32),
  )(Wt, x_pad, b)
  return out_t.T
